# baseline (speedup 1.0000x reference)
import functools

import jax
import jax.numpy as jnp
from jax import lax
from jax.experimental import pallas as pl
from jax.experimental.pallas import tpu as pltpu
from jax.experimental.pallas import tpu_sc as plsc

B = 4
S = 4096
D = 1024

NC = 2
NS = 16
NW = NC * NS

CHUNK = 16384
WORDS_PER_W = S * D // NW
NCHUNKS = WORDS_PER_W // CHUNK


def _sc_body(x_hbm, pe_hbm, o_hbm, pe_v, x_v):
    c = lax.axis_index("c")
    s = lax.axis_index("s")
    wid = s * NC + c
    base = wid * WORDS_PER_W

    def chunk_body(i, _):
        off = base + i * CHUNK
        pltpu.sync_copy(pe_hbm.at[pl.ds(off, CHUNK)], pe_v)

        def batch_body(b, _):
            pltpu.sync_copy(x_hbm.at[b, pl.ds(off, CHUNK)], x_v)

            @plsc.parallel_loop(0, CHUNK, 16, unroll=8)
            def _add(j):
                x_v[pl.ds(j, 16)] = x_v[pl.ds(j, 16)] + pe_v[pl.ds(j, 16)]

            pltpu.sync_copy(x_v, o_hbm.at[b, pl.ds(off, CHUNK)])
            return 0

        lax.fori_loop(0, B, batch_body, 0)
        return 0

    lax.fori_loop(0, NCHUNKS, chunk_body, 0)


@functools.partial(jax.jit, donate_argnums=())
def _sc_add(x2, pe2):
    mesh = plsc.VectorSubcoreMesh(core_axis_name="c", subcore_axis_name="s")
    f = pl.kernel(
        _sc_body,
        out_type=jax.ShapeDtypeStruct((B, S * D), jnp.float32),
        mesh=mesh,
        scratch_types=[
            pltpu.VMEM((CHUNK,), jnp.float32),
            pltpu.VMEM((CHUNK,), jnp.float32),
        ],
    )
    return f(x2, pe2)


def kernel(x, pos_embedding):
    x2 = x.reshape(B, S * D)
    pe2 = pos_embedding.reshape(S * D)
    out = _sc_add(x2, pe2)
    return out.reshape(B, S, D)


# speedup vs baseline: 1.1306x; 1.1306x over previous
import functools

import jax
import jax.numpy as jnp
from jax import lax
from jax.experimental import pallas as pl
from jax.experimental.pallas import tpu as pltpu
from jax.experimental.pallas import tpu_sc as plsc

B = 4
S = 4096
D = 1024

NC = 2
NS = 16
NW = NC * NS

CHUNK = 16384
WORDS_PER_W = S * D // NW
NCHUNKS = WORDS_PER_W // CHUNK
NSTEPS = NCHUNKS * B


def _sc_body(x_hbm, pe_hbm, o_hbm, pe_v, x_v, pe_sem, in_sem, out_sem):
    c = lax.axis_index("c")
    s = lax.axis_index("s")
    wid = s * NC + c
    base = wid * WORDS_PER_W

    pltpu.async_copy(pe_hbm.at[pl.ds(base, CHUNK)], pe_v.at[0], pe_sem.at[0])
    pltpu.async_copy(x_hbm.at[0, pl.ds(base, CHUNK)], x_v.at[0], in_sem.at[0])

    def step(t, _):
        i = t // B
        b = t % B
        slot = lax.rem(t, 2)
        nxt = lax.rem(t + 1, 2)
        off = base + i * CHUNK

        @pl.when(jnp.logical_and(b == 0, i + 1 < NCHUNKS))
        def _():
            off2 = base + (i + 1) * CHUNK
            pltpu.async_copy(
                pe_hbm.at[pl.ds(off2, CHUNK)],
                pe_v.at[lax.rem(i + 1, 2)],
                pe_sem.at[lax.rem(i + 1, 2)],
            )

        @pl.when(t + 1 < NSTEPS)
        def _():
            t2 = t + 1
            i2 = t2 // B
            b2 = t2 % B
            off2 = base + i2 * CHUNK

            @pl.when(t >= 1)
            def _():
                pltpu.make_async_copy(
                    x_v.at[nxt], o_hbm.at[0, pl.ds(base, CHUNK)], out_sem.at[nxt]
                ).wait()

            pltpu.async_copy(
                x_hbm.at[b2, pl.ds(off2, CHUNK)], x_v.at[nxt], in_sem.at[nxt]
            )

        @pl.when(b == 0)
        def _():
            pltpu.make_async_copy(
                pe_hbm.at[pl.ds(off, CHUNK)],
                pe_v.at[lax.rem(i, 2)],
                pe_sem.at[lax.rem(i, 2)],
            ).wait()

        pltpu.make_async_copy(
            x_hbm.at[b, pl.ds(off, CHUNK)], x_v.at[slot], in_sem.at[slot]
        ).wait()

        pslot = lax.rem(i, 2)

        @plsc.parallel_loop(0, CHUNK, 16, unroll=8)
        def _add(j):
            x_v[slot, pl.ds(j, 16)] = x_v[slot, pl.ds(j, 16)] + pe_v[pslot, pl.ds(j, 16)]

        pltpu.async_copy(x_v.at[slot], o_hbm.at[b, pl.ds(off, CHUNK)], out_sem.at[slot])
        return 0

    lax.fori_loop(0, NSTEPS, step, 0)

    pltpu.make_async_copy(
        x_v.at[0], o_hbm.at[0, pl.ds(base, CHUNK)], out_sem.at[0]
    ).wait()
    pltpu.make_async_copy(
        x_v.at[1], o_hbm.at[0, pl.ds(base, CHUNK)], out_sem.at[1]
    ).wait()


@jax.jit
def _sc_add(x2, pe2):
    mesh = plsc.VectorSubcoreMesh(core_axis_name="c", subcore_axis_name="s")
    f = pl.kernel(
        _sc_body,
        out_type=jax.ShapeDtypeStruct((B, S * D), jnp.float32),
        mesh=mesh,
        scratch_types=[
            pltpu.VMEM((2, CHUNK), jnp.float32),
            pltpu.VMEM((2, CHUNK), jnp.float32),
            pltpu.SemaphoreType.DMA((2,)),
            pltpu.SemaphoreType.DMA((2,)),
            pltpu.SemaphoreType.DMA((2,)),
        ],
    )
    return f(x2, pe2)


def kernel(x, pos_embedding):
    x2 = x.reshape(B, S * D)
    pe2 = pos_embedding.reshape(S * D)
    out = _sc_add(x2, pe2)
    return out.reshape(B, S, D)


# speedup vs baseline: 3.0862x; 2.7297x over previous
import jax
import jax.numpy as jnp
from jax import lax
from jax.experimental import pallas as pl
from jax.experimental.pallas import tpu as pltpu
from jax.experimental.pallas import tpu_sc as plsc

B = 4
S = 4096
D = 1024

NC = 2
NS = 16
NW = NC * NS

R = 16
ROWS_PER_W = S // NW
NCHUNKS = ROWS_PER_W // R
NSTEPS = NCHUNKS * B


def _sc_body(x_hbm, pe_hbm, o_hbm, pe_v, x_v, pe_sem, in_sem, out_sem):
    c = lax.axis_index("c")
    s = lax.axis_index("s")
    wid = s * NC + c
    base = wid * ROWS_PER_W

    pltpu.async_copy(pe_hbm.at[pl.ds(base, R)], pe_v.at[0], pe_sem.at[0])
    pltpu.async_copy(x_hbm.at[0, pl.ds(base, R)], x_v.at[0], in_sem.at[0])

    def step(t, _):
        i = t // B
        b = t % B
        slot = lax.rem(t, 2)
        nxt = lax.rem(t + 1, 2)
        row0 = base + i * R

        @pl.when(jnp.logical_and(b == 0, i + 1 < NCHUNKS))
        def _():
            pltpu.async_copy(
                pe_hbm.at[pl.ds(row0 + R, R)],
                pe_v.at[lax.rem(i + 1, 2)],
                pe_sem.at[lax.rem(i + 1, 2)],
            )

        @pl.when(t + 1 < NSTEPS)
        def _():
            t2 = t + 1
            i2 = t2 // B
            b2 = t2 % B
            row2 = base + i2 * R

            @pl.when(t >= 1)
            def _():
                pltpu.make_async_copy(
                    x_v.at[nxt], o_hbm.at[0, pl.ds(base, R)], out_sem.at[nxt]
                ).wait()

            pltpu.async_copy(
                x_hbm.at[b2, pl.ds(row2, R)], x_v.at[nxt], in_sem.at[nxt]
            )

        @pl.when(b == 0)
        def _():
            pltpu.make_async_copy(
                pe_hbm.at[pl.ds(row0, R)],
                pe_v.at[lax.rem(i, 2)],
                pe_sem.at[lax.rem(i, 2)],
            ).wait()

        pltpu.make_async_copy(
            x_hbm.at[b, pl.ds(row0, R)], x_v.at[slot], in_sem.at[slot]
        ).wait()

        pslot = lax.rem(i, 2)

        @plsc.parallel_loop(0, R * D, 16, unroll=8)
        def _add(j):
            r = j // D
            col = lax.rem(j, D)
            x_v[slot, r, pl.ds(col, 16)] = (
                x_v[slot, r, pl.ds(col, 16)] + pe_v[pslot, r, pl.ds(col, 16)]
            )

        pltpu.async_copy(x_v.at[slot], o_hbm.at[b, pl.ds(row0, R)], out_sem.at[slot])
        return 0

    lax.fori_loop(0, NSTEPS, step, 0)

    pltpu.make_async_copy(
        x_v.at[0], o_hbm.at[0, pl.ds(base, R)], out_sem.at[0]
    ).wait()
    pltpu.make_async_copy(
        x_v.at[1], o_hbm.at[0, pl.ds(base, R)], out_sem.at[1]
    ).wait()


@jax.jit
def _sc_add(x, pe):
    mesh = plsc.VectorSubcoreMesh(core_axis_name="c", subcore_axis_name="s")
    f = pl.kernel(
        _sc_body,
        out_type=jax.ShapeDtypeStruct((B, S, D), jnp.float32),
        mesh=mesh,
        scratch_types=[
            pltpu.VMEM((2, R, D), jnp.float32),
            pltpu.VMEM((2, R, D), jnp.float32),
            pltpu.SemaphoreType.DMA((2,)),
            pltpu.SemaphoreType.DMA((2,)),
            pltpu.SemaphoreType.DMA((2,)),
        ],
    )
    return f(x, pe)


def kernel(x, pos_embedding):
    return _sc_add(x, pos_embedding)


# speedup vs baseline: 3.1703x; 1.0273x over previous
import jax
import jax.numpy as jnp
from jax import lax
from jax.experimental import pallas as pl
from jax.experimental.pallas import tpu as pltpu
from jax.experimental.pallas import tpu_sc as plsc

B = 4
S = 4096
D = 1024

NC = 2
NS = 16
NW = NC * NS

R = 16
ROWS_PER_W = S // NW
NCHUNKS = ROWS_PER_W // R
NSTEPS = NCHUNKS * B


NBUF = 4


def _sc_body(x_hbm, pe_hbm, o_hbm, pe_v, x_v, pe_sem, in_sem, out_sem):
    c = lax.axis_index("c")
    s = lax.axis_index("s")
    wid = s * NC + c
    base = wid * ROWS_PER_W

    pltpu.async_copy(pe_hbm.at[pl.ds(base, R)], pe_v.at[0], pe_sem.at[0])
    for t0 in range(NBUF - 1):
        i0 = t0 // B
        b0 = t0 % B
        pltpu.async_copy(
            x_hbm.at[b0, pl.ds(base + i0 * R, R)], x_v.at[t0], in_sem.at[t0]
        )

    def step(t, _):
        i = t // B
        b = t % B
        slot = lax.rem(t, NBUF)
        row0 = base + i * R

        @pl.when(jnp.logical_and(b == 0, i + 1 < NCHUNKS))
        def _():
            pltpu.async_copy(
                pe_hbm.at[pl.ds(row0 + R, R)],
                pe_v.at[lax.rem(i + 1, 2)],
                pe_sem.at[lax.rem(i + 1, 2)],
            )

        @pl.when(t + NBUF - 1 < NSTEPS)
        def _():
            t2 = t + NBUF - 1
            i2 = t2 // B
            b2 = t2 % B
            s2 = lax.rem(t2, NBUF)

            @pl.when(t >= 1)
            def _():
                pltpu.make_async_copy(
                    x_v.at[s2], o_hbm.at[0, pl.ds(base, R)], out_sem.at[s2]
                ).wait()

            pltpu.async_copy(
                x_hbm.at[b2, pl.ds(base + i2 * R, R)], x_v.at[s2], in_sem.at[s2]
            )

        @pl.when(b == 0)
        def _():
            pltpu.make_async_copy(
                pe_hbm.at[pl.ds(row0, R)],
                pe_v.at[lax.rem(i, 2)],
                pe_sem.at[lax.rem(i, 2)],
            ).wait()

        pltpu.make_async_copy(
            x_hbm.at[b, pl.ds(row0, R)], x_v.at[slot], in_sem.at[slot]
        ).wait()

        pslot = lax.rem(i, 2)

        @plsc.parallel_loop(0, R * D, 16, unroll=8)
        def _add(j):
            r = j // D
            col = lax.rem(j, D)
            x_v[slot, r, pl.ds(col, 16)] = (
                x_v[slot, r, pl.ds(col, 16)] + pe_v[pslot, r, pl.ds(col, 16)]
            )

        pltpu.async_copy(x_v.at[slot], o_hbm.at[b, pl.ds(row0, R)], out_sem.at[slot])
        return 0

    lax.fori_loop(0, NSTEPS, step, 0)

    for k in range(NBUF):
        pltpu.make_async_copy(
            x_v.at[k], o_hbm.at[0, pl.ds(base, R)], out_sem.at[k]
        ).wait()


@jax.jit
def _sc_add(x, pe):
    mesh = plsc.VectorSubcoreMesh(core_axis_name="c", subcore_axis_name="s")
    f = pl.kernel(
        _sc_body,
        out_type=jax.ShapeDtypeStruct((B, S, D), jnp.float32),
        mesh=mesh,
        scratch_types=[
            pltpu.VMEM((2, R, D), jnp.float32),
            pltpu.VMEM((NBUF, R, D), jnp.float32),
            pltpu.SemaphoreType.DMA((2,)),
            pltpu.SemaphoreType.DMA((NBUF,)),
            pltpu.SemaphoreType.DMA((NBUF,)),
        ],
    )
    return f(x, pe)


def kernel(x, pos_embedding):
    return _sc_add(x, pos_embedding)


# speedup vs baseline: 3.5195x; 1.1101x over previous
import jax
import jax.numpy as jnp
from jax import lax
from jax.experimental import pallas as pl
from jax.experimental.pallas import tpu as pltpu
from jax.experimental.pallas import tpu_sc as plsc

B = 4
S = 4096
D = 1024

NC = 2
NS = 16
NW = NC * NS

R = 16
ROWS_PER_W = S // NW
NCHUNKS = ROWS_PER_W // R
NSTEPS = NCHUNKS * B


NBUF = 5
PD = 3


def _sc_body(x_hbm, pe_hbm, o_hbm, pe_v, x_v, pe_sem, in_sem, out_sem):
    c = lax.axis_index("c")
    s = lax.axis_index("s")
    wid = s * NC + c
    base = wid * ROWS_PER_W

    pltpu.async_copy(pe_hbm.at[pl.ds(base, R)], pe_v.at[0], pe_sem.at[0])
    for t0 in range(PD):
        i0 = t0 // B
        b0 = t0 % B
        pltpu.async_copy(
            x_hbm.at[b0, pl.ds(base + i0 * R, R)], x_v.at[t0], in_sem.at[t0]
        )

    def step(t, _):
        i = t // B
        b = t % B
        slot = lax.rem(t, NBUF)
        row0 = base + i * R

        @pl.when(jnp.logical_and(b == 0, i + 1 < NCHUNKS))
        def _():
            pltpu.async_copy(
                pe_hbm.at[pl.ds(row0 + R, R)],
                pe_v.at[lax.rem(i + 1, 2)],
                pe_sem.at[lax.rem(i + 1, 2)],
            )

        @pl.when(t + PD < NSTEPS)
        def _():
            t2 = t + PD
            i2 = t2 // B
            b2 = t2 % B
            s2 = lax.rem(t2, NBUF)

            @pl.when(t >= NBUF - PD)
            def _():
                pltpu.make_async_copy(
                    x_v.at[s2], o_hbm.at[0, pl.ds(base, R)], out_sem.at[s2]
                ).wait()

            pltpu.async_copy(
                x_hbm.at[b2, pl.ds(base + i2 * R, R)], x_v.at[s2], in_sem.at[s2]
            )

        @pl.when(b == 0)
        def _():
            pltpu.make_async_copy(
                pe_hbm.at[pl.ds(row0, R)],
                pe_v.at[lax.rem(i, 2)],
                pe_sem.at[lax.rem(i, 2)],
            ).wait()

        pltpu.make_async_copy(
            x_hbm.at[b, pl.ds(row0, R)], x_v.at[slot], in_sem.at[slot]
        ).wait()

        pslot = lax.rem(i, 2)

        @plsc.parallel_loop(0, R * D, 16, unroll=8)
        def _add(j):
            r = j // D
            col = lax.rem(j, D)
            x_v[slot, r, pl.ds(col, 16)] = (
                x_v[slot, r, pl.ds(col, 16)] + pe_v[pslot, r, pl.ds(col, 16)]
            )

        pltpu.async_copy(x_v.at[slot], o_hbm.at[b, pl.ds(row0, R)], out_sem.at[slot])
        return 0

    lax.fori_loop(0, NSTEPS, step, 0)

    for k in range(NBUF):
        pltpu.make_async_copy(
            x_v.at[k], o_hbm.at[0, pl.ds(base, R)], out_sem.at[k]
        ).wait()


@jax.jit
def _sc_add(x, pe):
    mesh = plsc.VectorSubcoreMesh(core_axis_name="c", subcore_axis_name="s")
    f = pl.kernel(
        _sc_body,
        out_type=jax.ShapeDtypeStruct((B, S, D), jnp.float32),
        mesh=mesh,
        scratch_types=[
            pltpu.VMEM((2, R, D), jnp.float32),
            pltpu.VMEM((NBUF, R, D), jnp.float32),
            pltpu.SemaphoreType.DMA((2,)),
            pltpu.SemaphoreType.DMA((NBUF,)),
            pltpu.SemaphoreType.DMA((NBUF,)),
        ],
    )
    return f(x, pe)


def kernel(x, pos_embedding):
    return _sc_add(x, pos_embedding)


# speedup vs baseline: 3.5843x; 1.0184x over previous
import jax
import jax.numpy as jnp
from jax import lax
from jax.experimental import pallas as pl
from jax.experimental.pallas import tpu as pltpu
from jax.experimental.pallas import tpu_sc as plsc

B = 4
S = 4096
D = 1024

NC = 2
NS = 16
NW = NC * NS

R = 8
ROWS_PER_W = S // NW
NCHUNKS = ROWS_PER_W // R
NSTEPS = NCHUNKS * B


NBUF = 10
PD = 5


def _sc_body(x_hbm, pe_hbm, o_hbm, pe_v, x_v, pe_sem, in_sem, out_sem):
    c = lax.axis_index("c")
    s = lax.axis_index("s")
    wid = s * NC + c
    base = wid * ROWS_PER_W

    pltpu.async_copy(pe_hbm.at[pl.ds(base, R)], pe_v.at[0], pe_sem.at[0])
    for t0 in range(PD):
        i0 = t0 // B
        b0 = t0 % B
        pltpu.async_copy(
            x_hbm.at[b0, pl.ds(base + i0 * R, R)], x_v.at[t0], in_sem.at[t0]
        )

    def step(t, _):
        i = t // B
        b = t % B
        slot = lax.rem(t, NBUF)
        row0 = base + i * R

        @pl.when(jnp.logical_and(b == 0, i + 1 < NCHUNKS))
        def _():
            pltpu.async_copy(
                pe_hbm.at[pl.ds(row0 + R, R)],
                pe_v.at[lax.rem(i + 1, 2)],
                pe_sem.at[lax.rem(i + 1, 2)],
            )

        @pl.when(t + PD < NSTEPS)
        def _():
            t2 = t + PD
            i2 = t2 // B
            b2 = t2 % B
            s2 = lax.rem(t2, NBUF)

            @pl.when(t >= NBUF - PD)
            def _():
                pltpu.make_async_copy(
                    x_v.at[s2], o_hbm.at[0, pl.ds(base, R)], out_sem.at[s2]
                ).wait()

            pltpu.async_copy(
                x_hbm.at[b2, pl.ds(base + i2 * R, R)], x_v.at[s2], in_sem.at[s2]
            )

        @pl.when(b == 0)
        def _():
            pltpu.make_async_copy(
                pe_hbm.at[pl.ds(row0, R)],
                pe_v.at[lax.rem(i, 2)],
                pe_sem.at[lax.rem(i, 2)],
            ).wait()

        pltpu.make_async_copy(
            x_hbm.at[b, pl.ds(row0, R)], x_v.at[slot], in_sem.at[slot]
        ).wait()

        pslot = lax.rem(i, 2)

        @plsc.parallel_loop(0, R * D, 16, unroll=8)
        def _add(j):
            r = j // D
            col = lax.rem(j, D)
            x_v[slot, r, pl.ds(col, 16)] = (
                x_v[slot, r, pl.ds(col, 16)] + pe_v[pslot, r, pl.ds(col, 16)]
            )

        pltpu.async_copy(x_v.at[slot], o_hbm.at[b, pl.ds(row0, R)], out_sem.at[slot])
        return 0

    lax.fori_loop(0, NSTEPS, step, 0)

    for k in range(NBUF):
        pltpu.make_async_copy(
            x_v.at[k], o_hbm.at[0, pl.ds(base, R)], out_sem.at[k]
        ).wait()


@jax.jit
def _sc_add(x, pe):
    mesh = plsc.VectorSubcoreMesh(core_axis_name="c", subcore_axis_name="s")
    f = pl.kernel(
        _sc_body,
        out_type=jax.ShapeDtypeStruct((B, S, D), jnp.float32),
        mesh=mesh,
        scratch_types=[
            pltpu.VMEM((2, R, D), jnp.float32),
            pltpu.VMEM((NBUF, R, D), jnp.float32),
            pltpu.SemaphoreType.DMA((2,)),
            pltpu.SemaphoreType.DMA((NBUF,)),
            pltpu.SemaphoreType.DMA((NBUF,)),
        ],
    )
    return f(x, pe)


def kernel(x, pos_embedding):
    return _sc_add(x, pos_embedding)
